# baseline (speedup 1.0000x reference)
import functools

import jax
import jax.numpy as jnp
from jax import lax
from jax.experimental import pallas as pl
from jax.experimental.pallas import tpu as pltpu
from jax.experimental.pallas import tpu_sc as plsc

_B = 16384
_NC = 2
_NS = 16
_HALF = _B // _NC
_CH = 2048
_NCHUNK = _HALF // _CH
_VR = 16
_FR = 100000
_CR = 1008

_PASSES = (
    (0, (0, 16)),
    (1, (32, 48)),
    (11, (208,)),
) + tuple((2 + j, (64 + 16 * j,)) for j in range(9))


def _encoder_body(idx_hbm, wxt, wyt, wft, wct, out_hbm,
                  wf_row, wx_rows, wy_rows, wc_row,
                  idx_bufs, out_bufs, sem_wf, sem_small, sem_i, sem_o):
    c = lax.axis_index("c")
    s = lax.axis_index("s")
    bbase = c * _HALF

    wf_ld = pltpu.async_copy(wft.at[s], wf_row, sem_wf)
    small_lds = [
        pltpu.async_copy(wxt.at[s], wx_rows.at[0], sem_small),
        pltpu.async_copy(wxt.at[_NS + s], wx_rows.at[1], sem_small),
        pltpu.async_copy(wyt.at[s], wy_rows.at[0], sem_small),
        pltpu.async_copy(wyt.at[_NS + s], wy_rows.at[1], sem_small),
        pltpu.async_copy(wct.at[s], wc_row, sem_small),
    ]

    tables = {0: ((wx_rows, 0), (wx_rows, 1)),
              1: ((wy_rows, 0), (wy_rows, 1)),
              11: ((wc_row, None),)}
    for j in range(9):
        tables[2 + j] = ((wf_row, None),)

    steps = [(pi, ch) for pi in range(len(_PASSES)) for ch in range(_NCHUNK)]

    def idx_copy(step, buf):
        irow = _PASSES[steps[step][0]][0]
        ch = steps[step][1]
        return pltpu.async_copy(
            idx_hbm.at[irow, pl.ds(bbase + ch * _CH, _CH)],
            idx_bufs.at[buf], sem_i)

    for ld in small_lds:
        ld.wait()

    pending_idx = idx_copy(0, 0)
    pending_out = [None, None, None, None]
    waited_wf = False
    for step, (pi, ch) in enumerate(steps):
        irow, bases = _PASSES[pi]
        if irow >= 2 and irow != 11 and not waited_wf:
            wf_ld.wait()
            waited_wf = True
        buf = step % 2
        pending_idx.wait()
        if step + 1 < len(steps):
            pending_idx = idx_copy(step + 1, 1 - buf)

        idxc = idx_bufs.at[buf]
        targets = tables[irow]
        outcs = []
        for t in range(len(targets)):
            ob = 2 * buf + t
            if pending_out[ob] is not None:
                pending_out[ob].wait()
                pending_out[ob] = None
            outcs.append(out_bufs.at[ob])

        @plsc.parallel_loop(0, _CH // 128, 1)
        def q_body(q, idxc=idxc, targets=targets, outcs=outcs):
            for p in range(8):
                iv = idxc[pl.ds(q * 128 + p * _VR, _VR)]
                for t, (row_ref, sub) in enumerate(targets):
                    if sub is None:
                        val = plsc.load_gather(row_ref, [iv])
                    else:
                        val = plsc.load_gather(
                            row_ref,
                            [jnp.full((_VR,), sub, jnp.int32), iv])
                    outcs[t][q, pl.ds(p * _VR, _VR)] = val

        for t, base in enumerate(bases):
            r = base + s
            ob = 2 * buf + t
            pending_out[ob] = pltpu.async_copy(
                out_bufs.at[ob],
                out_hbm.at[r // 8,
                           pl.ds(c * (_HALF // 128) + ch * (_CH // 128),
                                 _CH // 128), r % 8],
                sem_o.at[ob])
    for po in pending_out:
        if po is not None:
            po.wait()


@functools.partial(
    pl.kernel,
    mesh=plsc.VectorSubcoreMesh(core_axis_name="c", subcore_axis_name="s"),
    out_type=jax.ShapeDtypeStruct((224 // 8, _B // 128, 8, 128), jnp.float32),
    scratch_types=[
        pltpu.VMEM((_FR,), jnp.float32),
        pltpu.VMEM((2, 1024), jnp.float32),
        pltpu.VMEM((2, 1024), jnp.float32),
        pltpu.VMEM((_CR,), jnp.float32),
        pltpu.VMEM((2, _CH), jnp.int32),
        pltpu.VMEM((4, _CH // 128, 128), jnp.float32),
        pltpu.SemaphoreType.DMA,
        pltpu.SemaphoreType.DMA,
        pltpu.SemaphoreType.DMA,
        pltpu.SemaphoreType.DMA((4,)),
    ],
    compiler_params=pltpu.CompilerParams(
        use_tc_tiling_on_sc=False, needs_layout_passes=False),
)
def _encoder(*args):
    _encoder_body(*args)


def kernel(coords, obs, n_completed, Wx, Wy, Wf, Wc):
    b = coords.shape[0]
    cx = coords[:, 0].astype(jnp.int32)
    cy = coords[:, 1].astype(jnp.int32)
    obs_t = obs.reshape(b, 9).T.astype(jnp.int32)
    nc = n_completed.reshape(-1).astype(jnp.int32)
    idx = jnp.concatenate([cx[None], cy[None], obs_t, nc[None]], axis=0)
    wxt = Wx.T
    wyt = Wy.T
    wft = Wf.T
    wct = jnp.pad(Wc.T, ((0, 0), (0, _CR - Wc.shape[0])))
    out4 = _encoder(idx, wxt, wyt, wft, wct)
    return out4.transpose(0, 2, 1, 3).reshape(224, b).T


# speedup vs baseline: 1.2302x; 1.2302x over previous
import functools

import jax
import jax.numpy as jnp
from jax import lax
from jax.experimental import pallas as pl
from jax.experimental.pallas import tpu as pltpu
from jax.experimental.pallas import tpu_sc as plsc

_B = 16384
_NC = 2
_NS = 16
_HALF = _B // _NC
_CH = 4096
_NCHUNK = _HALF // _CH
_VR = 16
_FR = 100000
_CR = 1008

_PASSES = (
    (0, 0, (0, 16)),
    (0, 1, (32, 48)),
    (2, 0, (208,)),
) + tuple((1, j, (64 + 16 * j,)) for j in range(9))


def _encoder_body(coords_t, obs_t, nc_t, wxt, wyt, wft, wct, out_hbm,
                  wf_row, wx_rows, wy_rows, wc_row,
                  idx_bufs, out_bufs, sem_wf, sem_small, sem_i, sem_o):
    c = lax.axis_index("c")
    s = lax.axis_index("s")
    bbase = c * _HALF
    idx_srcs = (coords_t, obs_t, nc_t)

    wf_ld = pltpu.async_copy(wft.at[s], wf_row, sem_wf)
    small_lds = [
        pltpu.async_copy(wxt.at[s], wx_rows.at[0], sem_small),
        pltpu.async_copy(wxt.at[_NS + s], wx_rows.at[1], sem_small),
        pltpu.async_copy(wyt.at[s], wy_rows.at[0], sem_small),
        pltpu.async_copy(wyt.at[_NS + s], wy_rows.at[1], sem_small),
        pltpu.async_copy(wct.at[s], wc_row, sem_small),
    ]

    targets_by_pass = [((wx_rows, 0), (wx_rows, 1)),
                       ((wy_rows, 0), (wy_rows, 1)),
                       ((wc_row, None),)] + [((wf_row, None),)] * 9

    steps = [(pi, ch) for pi in range(len(_PASSES)) for ch in range(_NCHUNK)]

    def idx_copy(step, buf):
        src, irow, _ = _PASSES[steps[step][0]]
        ch = steps[step][1]
        return pltpu.async_copy(
            idx_srcs[src].at[irow, pl.ds(bbase + ch * _CH, _CH)],
            idx_bufs.at[buf], sem_i)

    for ld in small_lds:
        ld.wait()

    pending_idx = idx_copy(0, 0)
    pending_out = [None, None, None, None]
    waited_wf = False
    for step, (pi, ch) in enumerate(steps):
        _, _, bases = _PASSES[pi]
        if pi >= 3 and not waited_wf:
            wf_ld.wait()
            waited_wf = True
        buf = step % 2
        pending_idx.wait()
        if step + 1 < len(steps):
            pending_idx = idx_copy(step + 1, 1 - buf)

        idxc = idx_bufs.at[buf]
        targets = targets_by_pass[pi]
        outcs = []
        for t in range(len(targets)):
            ob = 2 * buf + t
            if pending_out[ob] is not None:
                pending_out[ob].wait()
                pending_out[ob] = None
            outcs.append(out_bufs.at[ob])

        @plsc.parallel_loop(0, _CH // 128, 1)
        def q_body(q, idxc=idxc, targets=targets, outcs=outcs):
            for p in range(8):
                iv = idxc[pl.ds(q * 128 + p * _VR, _VR)]
                for t, (row_ref, sub) in enumerate(targets):
                    if sub is None:
                        val = plsc.load_gather(row_ref, [iv])
                    else:
                        val = plsc.load_gather(
                            row_ref,
                            [jnp.full((_VR,), sub, jnp.int32), iv])
                    outcs[t][q, pl.ds(p * _VR, _VR)] = val

        for t, base in enumerate(bases):
            r = base + s
            ob = 2 * buf + t
            pending_out[ob] = pltpu.async_copy(
                out_bufs.at[ob],
                out_hbm.at[r // 8,
                           pl.ds(c * (_HALF // 128) + ch * (_CH // 128),
                                 _CH // 128), r % 8],
                sem_o.at[ob])
    for po in pending_out:
        if po is not None:
            po.wait()


@functools.partial(
    pl.kernel,
    mesh=plsc.VectorSubcoreMesh(core_axis_name="c", subcore_axis_name="s"),
    out_type=jax.ShapeDtypeStruct((224 // 8, _B // 128, 8, 128), jnp.float32),
    scratch_types=[
        pltpu.VMEM((_FR,), jnp.float32),
        pltpu.VMEM((2, 1024), jnp.float32),
        pltpu.VMEM((2, 1024), jnp.float32),
        pltpu.VMEM((_CR,), jnp.float32),
        pltpu.VMEM((2, _CH), jnp.int32),
        pltpu.VMEM((4, _CH // 128, 128), jnp.float32),
        pltpu.SemaphoreType.DMA,
        pltpu.SemaphoreType.DMA,
        pltpu.SemaphoreType.DMA,
        pltpu.SemaphoreType.DMA((4,)),
    ],
    compiler_params=pltpu.CompilerParams(
        use_tc_tiling_on_sc=False, needs_layout_passes=False),
)
def _encoder(*args):
    _encoder_body(*args)


def kernel(coords, obs, n_completed, Wx, Wy, Wf, Wc):
    b = coords.shape[0]
    coords_t = coords.T.astype(jnp.int32)
    obs_t = obs.reshape(b, 9).T.astype(jnp.int32)
    nc_t = n_completed.reshape(b, 1).T.astype(jnp.int32)
    wxt = Wx.T
    wyt = Wy.T
    wft = Wf.T
    wct = jnp.pad(Wc.T, ((0, 0), (0, _CR - Wc.shape[0])))
    out4 = _encoder(coords_t, obs_t, nc_t, wxt, wyt, wft, wct)
    return out4.transpose(0, 2, 1, 3).reshape(224, b).T
